# Initial kernel scaffold; baseline (speedup 1.0000x reference)
import functools

import jax
import jax.numpy as jnp
from jax import lax
from jax.experimental import pallas as pl
from jax.experimental.pallas import tpu as pltpu
from jax.experimental.pallas import tpu_sc as plsc

N = 10000
G = 64
NP = 10240
RPS = NP // 16
B = 128
E = 320000
EP = 323584
NCH = EP // (16 * B)
DCH = EP // (32 * B)
BLK = 256
NBLK = NP // BLK

_mesh = plsc.VectorSubcoreMesh(core_axis_name="c", subcore_axis_name="s")


def _sc_degree(dstd):
    @functools.partial(
        pl.kernel,
        mesh=_mesh,
        out_type=jax.ShapeDtypeStruct((2, NP, 16), jnp.float32),
        scratch_types=[
            pltpu.VMEM((DCH, B), jnp.int32),
            pltpu.VMEM((B, 16), jnp.float32),
            pltpu.VMEM_SHARED((NP, 16), jnp.float32),
        ],
    )
    def k(dstd_hbm, out_hbm, idx_v, ones_v, acc):
        c = lax.axis_index("c")
        s = lax.axis_index("s")
        r0 = s * RPS
        zero16 = jnp.zeros((16,), jnp.float32)

        @pl.loop(0, B)
        def _(i):
            ones_v[i] = zero16

        @pl.loop(0, RPS // B)
        def _(kk):
            pltpu.sync_copy(ones_v, acc.at[pl.ds(r0 + kk * B, B)])

        one16 = jnp.ones((16,), jnp.float32)

        @pl.loop(0, B)
        def _(i):
            ones_v[i] = one16

        pltpu.sync_copy(dstd_hbm.at[c, s], idx_v)
        plsc.subcore_barrier()

        @pl.loop(0, DCH)
        def _(j):
            pltpu.sync_copy(ones_v, acc.at[idx_v.at[j]], add=True)

        plsc.subcore_barrier()
        pltpu.sync_copy(acc.at[pl.ds(r0, RPS)], out_hbm.at[c, pl.ds(r0, RPS)])

    return k(dstd)


def _make_sc_agg(Dh):
    @functools.partial(
        pl.kernel,
        mesh=_mesh,
        out_type=jax.ShapeDtypeStruct((2 * NP, Dh), jnp.float32),
        scratch_types=[
            pltpu.VMEM((NCH, B), jnp.int32),
            pltpu.VMEM((NCH, B), jnp.int32),
            pltpu.VMEM((B, Dh), jnp.float32),
            pltpu.VMEM((B, Dh), jnp.float32),
            pltpu.VMEM_SHARED((NP, Dh), jnp.float32),
            pltpu.SemaphoreType.DMA,
            pltpu.SemaphoreType.DMA,
        ],
    )
    def k(table_hbm, src_hbm, dst_hbm, out_hbm,
          src_v, dst_v, row0, row1, acc, sem0, sem1):
        c = lax.axis_index("c")
        s = lax.axis_index("s")
        r0 = s * RPS
        base = c * NP
        pltpu.sync_copy(table_hbm.at[pl.ds(base + r0, RPS)],
                        acc.at[pl.ds(r0, RPS)])
        pltpu.sync_copy(src_hbm.at[c, s], src_v)
        pltpu.sync_copy(dst_hbm.at[s], dst_v)
        plsc.subcore_barrier()

        pltpu.async_copy(table_hbm.at[src_v.at[0]], row0, sem0)

        @pl.loop(0, NCH, step=2)
        def _(j):
            pltpu.async_copy(table_hbm.at[src_v.at[j + 1]], row1, sem1)
            pltpu.make_async_copy(table_hbm.at[src_v.at[j]], row0, sem0).wait()
            pltpu.sync_copy(row0, acc.at[dst_v.at[j]], add=True)

            @pl.when(j + 2 < NCH)
            def _():
                pltpu.async_copy(table_hbm.at[src_v.at[j + 2]], row0, sem0)

            pltpu.make_async_copy(table_hbm.at[src_v.at[j + 1]], row1,
                                  sem1).wait()
            pltpu.sync_copy(row1, acc.at[dst_v.at[j + 1]], add=True)

        plsc.subcore_barrier()
        pltpu.sync_copy(acc.at[pl.ds(r0, RPS)],
                        out_hbm.at[pl.ds(base + r0, RPS)])

    return k


_sc_agg64 = _make_sc_agg(64)
_sc_agg128 = _make_sc_agg(128)


def _dis_block(deg_ref, i):
    deg = deg_ref[0, :, 0:1] + deg_ref[1, :, 0:1] + 1.0
    dis = lax.rsqrt(deg)
    rows = i * BLK + lax.broadcasted_iota(jnp.int32, (BLK, 1), 0)
    return jnp.where(rows < N, dis, 0.0)


def _tc_prep(degp, x_pad):
    def body(deg_ref, x_ref, o_ref):
        dis = _dis_block(deg_ref, pl.program_id(0))
        xb = x_ref[...]
        o_ref[0] = dis * xb[:, :64]
        o_ref[1] = dis * xb[:, 64:]

    return pl.pallas_call(
        body,
        grid=(NBLK,),
        in_specs=[pl.BlockSpec((2, BLK, 16), lambda i: (0, i, 0)),
                  pl.BlockSpec((BLK, 128), lambda i: (i, 0))],
        out_specs=pl.BlockSpec((2, BLK, 64), lambda i: (0, i, 0)),
        out_shape=jax.ShapeDtypeStruct((2, NP, 64), jnp.float32),
    )(degp, x_pad)


def _matmul(a, w_ref, b_ref):
    y = lax.dot_general(a, w_ref[...], (((1,), (0,)), ((), ())),
                        preferred_element_type=jnp.float32,
                        precision=lax.Precision.HIGHEST)
    return y + b_ref[...]


def _tc_layer(agg, degp, W, b):
    Din = W.shape[0]
    Dhi = Din // 2

    def body(agg_ref, deg_ref, w_ref, b_ref, o_ref):
        dis = _dis_block(deg_ref, pl.program_id(0))
        full = jnp.concatenate([agg_ref[0], agg_ref[1]], axis=1)
        t = jnp.maximum(_matmul(dis * full, w_ref, b_ref), 0.0)
        o_ref[0] = dis * t[:, :128]
        o_ref[1] = dis * t[:, 128:]

    return pl.pallas_call(
        body,
        grid=(NBLK,),
        in_specs=[pl.BlockSpec((2, BLK, Dhi), lambda i: (0, i, 0)),
                  pl.BlockSpec((2, BLK, 16), lambda i: (0, i, 0)),
                  pl.BlockSpec(W.shape, lambda i: (0, 0)),
                  pl.BlockSpec((1, 256), lambda i: (0, 0))],
        out_specs=pl.BlockSpec((2, BLK, 128), lambda i: (0, i, 0)),
        out_shape=jax.ShapeDtypeStruct((2, NP, 128), jnp.float32),
    )(agg, degp, W, b.reshape(1, -1))


def _tc_final(agg, degp, W3, b3, batch3, lin_W, lin_b):
    def body(agg_ref, deg_ref, w_ref, b_ref, bat_ref, lw_ref, lb_ref,
             o_ref, sums, counts):
        i = pl.program_id(0)
        dis = _dis_block(deg_ref, i)
        full = jnp.concatenate([agg_ref[0], agg_ref[1]], axis=1)
        h3 = jnp.maximum(_matmul(dis * full, w_ref, b_ref), 0.0)
        bat = bat_ref[0, 0, :]
        gids = lax.broadcasted_iota(jnp.int32, (G, BLK), 0)
        onehot_t = (gids == bat[None, :]).astype(jnp.float32)
        ps = lax.dot_general(onehot_t, h3, (((1,), (0,)), ((), ())),
                             preferred_element_type=jnp.float32,
                             precision=lax.Precision.HIGHEST)
        pc = jnp.broadcast_to(jnp.sum(onehot_t, axis=1, keepdims=True),
                              (G, 128))

        @pl.when(i == 0)
        def _():
            sums[...] = ps
            counts[...] = pc

        @pl.when(i > 0)
        def _():
            sums[...] += ps
            counts[...] += pc

        @pl.when(i == NBLK - 1)
        def _():
            pooled = sums[...] / jnp.maximum(counts[:, 0:1], 1.0)
            o_ref[...] = lax.dot_general(
                pooled, lw_ref[...], (((1,), (0,)), ((), ())),
                preferred_element_type=jnp.float32,
                precision=lax.Precision.HIGHEST) + lb_ref[...]

    return pl.pallas_call(
        body,
        grid=(NBLK,),
        in_specs=[pl.BlockSpec((2, BLK, 128), lambda i: (0, i, 0)),
                  pl.BlockSpec((2, BLK, 16), lambda i: (0, i, 0)),
                  pl.BlockSpec(W3.shape, lambda i: (0, 0)),
                  pl.BlockSpec((1, 256), lambda i: (0, 0)),
                  pl.BlockSpec((1, 1, BLK), lambda i: (i, 0, 0)),
                  pl.BlockSpec(lin_W.shape, lambda i: (0, 0)),
                  pl.BlockSpec((1, 16), lambda i: (0, 0))],
        out_specs=pl.BlockSpec((G, 16), lambda i: (0, 0)),
        out_shape=jax.ShapeDtypeStruct((G, 16), jnp.float32),
        scratch_shapes=[pltpu.VMEM((G, 256), jnp.float32),
                        pltpu.VMEM((G, 128), jnp.float32)],
    )(agg, degp, W3, b3.reshape(1, -1), batch3, lin_W, lin_b.reshape(1, -1))


def kernel(x, edge_index, batch, W1, b1, W2, b2, W3, b3, lin_W, lin_b):
    src = edge_index[0]
    dst = edge_index[1]
    pad = EP - E
    src_p = jnp.concatenate([src, jnp.full((pad,), N, jnp.int32)])
    dst_p = jnp.concatenate([dst, jnp.full((pad,), N, jnp.int32)])
    src2 = (src_p.reshape(16, NCH, B)[None]
            + jnp.array([0, NP], jnp.int32)[:, None, None, None])
    dst2 = dst_p.reshape(16, NCH, B)
    dstd = dst_p.reshape(2, 16, DCH, B)
    x_pad = jnp.zeros((NP, 128), jnp.float32).at[:N].set(x)
    batch3 = jnp.concatenate(
        [batch, jnp.full((NP - N,), G, jnp.int32)]).reshape(NBLK, 1, BLK)

    degp = _sc_degree(dstd)
    table1 = _tc_prep(degp, x_pad)
    agg1 = _sc_agg64(table1.reshape(2 * NP, 64), src2, dst2)
    table2 = _tc_layer(agg1.reshape(2, NP, 64), degp, W1, b1)
    agg2 = _sc_agg128(table2.reshape(2 * NP, 128), src2, dst2)
    table3 = _tc_layer(agg2.reshape(2, NP, 128), degp, W2, b2)
    agg3 = _sc_agg128(table3.reshape(2 * NP, 128), src2, dst2)
    return _tc_final(agg3.reshape(2, NP, 128), degp, W3, b3,
                     batch3, lin_W, lin_b)


# speedup vs baseline: 8.9453x; 8.9453x over previous
import functools

import jax
import jax.numpy as jnp
from jax import lax
from jax.experimental import pallas as pl
from jax.experimental.pallas import tpu as pltpu
from jax.experimental.pallas import tpu_sc as plsc

N = 10000
G = 64
NP = 10240
RPS = NP // 16
B = 128
E = 320000
EP = 327680
NCH = EP // (16 * B)
DCH = EP // (32 * B)
D = 128
BLK = 256
NBLK = NP // BLK

_mesh = plsc.VectorSubcoreMesh(core_axis_name="c", subcore_axis_name="s")


def _sc_degree(dstd):
    @functools.partial(
        pl.kernel,
        mesh=_mesh,
        out_type=jax.ShapeDtypeStruct((2, NP, 16), jnp.float32),
        scratch_types=[
            pltpu.VMEM((DCH, B), jnp.int32),
            pltpu.VMEM((B, 16), jnp.float32),
            pltpu.VMEM_SHARED((NP, 16), jnp.float32),
        ],
        compiler_params=pltpu.CompilerParams(use_tc_tiling_on_sc=False),
    )
    def k(dstd_hbm, out_hbm, idx_v, ones_v, acc):
        c = lax.axis_index("c")
        s = lax.axis_index("s")
        r0 = s * RPS
        zero16 = jnp.zeros((16,), jnp.float32)

        @pl.loop(0, B)
        def _(i):
            ones_v[i] = zero16

        @pl.loop(0, RPS // B)
        def _(kk):
            pltpu.sync_copy(ones_v, acc.at[pl.ds(r0 + kk * B, B)])

        one16 = jnp.ones((16,), jnp.float32)

        @pl.loop(0, B)
        def _(i):
            ones_v[i] = one16

        pltpu.sync_copy(dstd_hbm.at[pl.ds((c * 16 + s) * DCH, DCH)], idx_v)
        plsc.subcore_barrier()

        @pl.loop(0, DCH)
        def _(j):
            pltpu.sync_copy(ones_v, acc.at[idx_v.at[j]], add=True)

        plsc.subcore_barrier()
        pltpu.sync_copy(acc.at[pl.ds(r0, RPS)], out_hbm.at[c, pl.ds(r0, RPS)])

    return k(dstd)


IB = 16


def _agg_loop(table_hbm, acc, src_hbm, dst_hbm, slab0, nch,
              srcb, dstb, row0, row1, sem0, sem1, isem):
    pltpu.sync_copy(src_hbm.at[pl.ds(slab0, IB)], srcb.at[0])
    pltpu.sync_copy(dst_hbm.at[pl.ds(slab0, IB)], dstb.at[0])
    pltpu.async_copy(table_hbm.at[srcb.at[0].at[0]], row0, sem0)

    @pl.loop(0, nch, step=2)
    def _(t):
        g = t // IB
        r = t - g * IB
        gb = g % 2
        nb = (g + 1) % 2

        @pl.when((r == 0) & (t + IB < nch))
        def _():
            pltpu.async_copy(src_hbm.at[pl.ds(slab0 + (g + 1) * IB, IB)],
                             srcb.at[nb], isem)
            pltpu.async_copy(dst_hbm.at[pl.ds(slab0 + (g + 1) * IB, IB)],
                             dstb.at[nb], isem)

        pltpu.async_copy(table_hbm.at[srcb.at[gb].at[r + 1]], row1, sem1)
        pltpu.make_async_copy(table_hbm.at[srcb.at[gb].at[r]], row0,
                              sem0).wait()
        pltpu.sync_copy(row0, acc.at[dstb.at[gb].at[r]], add=True)

        @pl.when(r == IB - 2)
        def _():
            @pl.when(t + 2 < nch)
            def _():
                pltpu.make_async_copy(
                    src_hbm.at[pl.ds(slab0 + (g + 1) * IB, IB)],
                    srcb.at[nb], isem).wait()
                pltpu.make_async_copy(
                    dst_hbm.at[pl.ds(slab0 + (g + 1) * IB, IB)],
                    dstb.at[nb], isem).wait()
                pltpu.async_copy(table_hbm.at[srcb.at[nb].at[0]], row0, sem0)

        @pl.when(r != IB - 2)
        def _():
            pltpu.async_copy(table_hbm.at[srcb.at[gb].at[r + 2]], row0, sem0)

        pltpu.make_async_copy(table_hbm.at[srcb.at[gb].at[r + 1]], row1,
                              sem1).wait()
        pltpu.sync_copy(row1, acc.at[dstb.at[gb].at[r + 1]], add=True)


def _sc_agg_cols(table, src2, dst2):
    @functools.partial(
        pl.kernel,
        mesh=_mesh,
        out_type=jax.ShapeDtypeStruct((2 * NP, D), jnp.float32),
        scratch_types=[
            pltpu.VMEM((2, IB, B), jnp.int32),
            pltpu.VMEM((2, IB, B), jnp.int32),
            pltpu.VMEM((B, D), jnp.float32),
            pltpu.VMEM((B, D), jnp.float32),
            pltpu.VMEM_SHARED((NP, D), jnp.float32),
            pltpu.SemaphoreType.DMA,
            pltpu.SemaphoreType.DMA,
            pltpu.SemaphoreType.DMA,
        ],
        compiler_params=pltpu.CompilerParams(use_tc_tiling_on_sc=False),
    )
    def k(table_hbm, src_hbm, dst_hbm, out_hbm,
          srcb, dstb, row0, row1, acc, sem0, sem1, isem):
        c = lax.axis_index("c")
        s = lax.axis_index("s")
        r0 = s * RPS
        base = c * NP
        pltpu.sync_copy(table_hbm.at[pl.ds(base + r0, RPS)],
                        acc.at[pl.ds(r0, RPS)])
        plsc.subcore_barrier()
        _agg_loop(table_hbm, acc,
                  src_hbm.at[pl.ds((c * 16 + s) * NCH, NCH)],
                  dst_hbm.at[pl.ds(s * NCH, NCH)],
                  0, NCH, srcb, dstb, row0, row1, sem0, sem1, isem)
        plsc.subcore_barrier()
        pltpu.sync_copy(acc.at[pl.ds(r0, RPS)],
                        out_hbm.at[pl.ds(base + r0, RPS)])

    return k(table, src2, dst2)


def _sc_agg_edges(table, srcd, dstd):
    @functools.partial(
        pl.kernel,
        mesh=_mesh,
        out_type=jax.ShapeDtypeStruct((2, NP, D), jnp.float32),
        scratch_types=[
            pltpu.VMEM((2, IB, B), jnp.int32),
            pltpu.VMEM((2, IB, B), jnp.int32),
            pltpu.VMEM((B, D), jnp.float32),
            pltpu.VMEM((B, D), jnp.float32),
            pltpu.VMEM_SHARED((NP, D), jnp.float32),
            pltpu.SemaphoreType.DMA,
            pltpu.SemaphoreType.DMA,
            pltpu.SemaphoreType.DMA,
        ],
        compiler_params=pltpu.CompilerParams(use_tc_tiling_on_sc=False),
    )
    def k(table_hbm, src_hbm, dst_hbm, out_hbm,
          srcb, dstb, row0, row1, acc, sem0, sem1, isem):
        c = lax.axis_index("c")
        s = lax.axis_index("s")
        r0 = s * RPS

        @pl.when(c == 0)
        def _():
            pltpu.sync_copy(table_hbm.at[pl.ds(r0, RPS)],
                            acc.at[pl.ds(r0, RPS)])

        @pl.when(c == 1)
        def _():
            zero16 = jnp.zeros((16,), jnp.float32)

            @pl.loop(0, B)
            def _(i):
                @pl.loop(0, D // 16)
                def _(kk):
                    row0[i, pl.ds(kk * 16, 16)] = zero16

            @pl.loop(0, RPS // B)
            def _(kk):
                pltpu.sync_copy(row0, acc.at[pl.ds(r0 + kk * B, B)])

        plsc.subcore_barrier()
        _agg_loop(table_hbm, acc,
                  src_hbm.at[pl.ds((c * 16 + s) * DCH, DCH)],
                  dst_hbm.at[pl.ds((c * 16 + s) * DCH, DCH)],
                  0, DCH, srcb, dstb, row0, row1, sem0, sem1, isem)
        plsc.subcore_barrier()
        pltpu.sync_copy(acc.at[pl.ds(r0, RPS)],
                        out_hbm.at[c, pl.ds(r0, RPS)])

    return k(table, srcd, dstd)


def _dis_block(deg_ref, i):
    deg = deg_ref[0, :, 0:1] + deg_ref[1, :, 0:1] + 1.0
    dis = lax.rsqrt(deg)
    rows = i * BLK + lax.broadcasted_iota(jnp.int32, (BLK, 1), 0)
    return jnp.where(rows < N, dis, 0.0)


def _tc_prep(degp, x_pad):
    def body(deg_ref, x_ref, o_ref):
        dis = _dis_block(deg_ref, pl.program_id(0))
        o_ref[...] = dis * x_ref[...]

    return pl.pallas_call(
        body,
        grid=(NBLK,),
        in_specs=[pl.BlockSpec((2, BLK, 16), lambda i: (0, i, 0)),
                  pl.BlockSpec((BLK, 128), lambda i: (i, 0))],
        out_specs=pl.BlockSpec((BLK, 128), lambda i: (i, 0)),
        out_shape=jax.ShapeDtypeStruct((NP, 128), jnp.float32),
    )(degp, x_pad)


def _matmul(a, w_ref, b_ref):
    y = lax.dot_general(a, w_ref[...], (((1,), (0,)), ((), ())),
                        preferred_element_type=jnp.float32,
                        precision=lax.Precision.HIGHEST)
    return y + b_ref[...]


def _combine(agg_ref, mode):
    if mode == "sum":
        return agg_ref[0] + agg_ref[1]
    return jnp.concatenate([agg_ref[0], agg_ref[1]], axis=1)


def _tc_layer(agg, degp, W, b, mode):
    def body(agg_ref, deg_ref, w_ref, b_ref, o_ref):
        dis = _dis_block(deg_ref, pl.program_id(0))
        t = jnp.maximum(
            _matmul(dis * _combine(agg_ref, mode), w_ref, b_ref), 0.0)
        o_ref[0] = dis * t[:, :128]
        o_ref[1] = dis * t[:, 128:]

    return pl.pallas_call(
        body,
        grid=(NBLK,),
        in_specs=[pl.BlockSpec((2, BLK, D), lambda i: (0, i, 0)),
                  pl.BlockSpec((2, BLK, 16), lambda i: (0, i, 0)),
                  pl.BlockSpec(W.shape, lambda i: (0, 0)),
                  pl.BlockSpec((1, 256), lambda i: (0, 0))],
        out_specs=pl.BlockSpec((2, BLK, 128), lambda i: (0, i, 0)),
        out_shape=jax.ShapeDtypeStruct((2, NP, 128), jnp.float32),
    )(agg, degp, W, b.reshape(1, -1))


def _tc_final(agg, degp, W3, b3, batch3, lin_W, lin_b):
    def body(agg_ref, deg_ref, w_ref, b_ref, bat_ref, lw_ref, lb_ref,
             o_ref, sums, counts):
        i = pl.program_id(0)
        dis = _dis_block(deg_ref, i)
        h3 = jnp.maximum(
            _matmul(dis * _combine(agg_ref, "concat"), w_ref, b_ref), 0.0)
        bat = bat_ref[0, 0, :]
        gids = lax.broadcasted_iota(jnp.int32, (G, BLK), 0)
        onehot_t = (gids == bat[None, :]).astype(jnp.float32)
        ps = lax.dot_general(onehot_t, h3, (((1,), (0,)), ((), ())),
                             preferred_element_type=jnp.float32,
                             precision=lax.Precision.HIGHEST)
        pc = jnp.broadcast_to(jnp.sum(onehot_t, axis=1, keepdims=True),
                              (G, 128))

        @pl.when(i == 0)
        def _():
            sums[...] = ps
            counts[...] = pc

        @pl.when(i > 0)
        def _():
            sums[...] += ps
            counts[...] += pc

        @pl.when(i == NBLK - 1)
        def _():
            pooled = sums[...] / jnp.maximum(counts[:, 0:1], 1.0)
            o_ref[...] = lax.dot_general(
                pooled, lw_ref[...], (((1,), (0,)), ((), ())),
                preferred_element_type=jnp.float32,
                precision=lax.Precision.HIGHEST) + lb_ref[...]

    return pl.pallas_call(
        body,
        grid=(NBLK,),
        in_specs=[pl.BlockSpec((2, BLK, D), lambda i: (0, i, 0)),
                  pl.BlockSpec((2, BLK, 16), lambda i: (0, i, 0)),
                  pl.BlockSpec(W3.shape, lambda i: (0, 0)),
                  pl.BlockSpec((1, 256), lambda i: (0, 0)),
                  pl.BlockSpec((1, 1, BLK), lambda i: (i, 0, 0)),
                  pl.BlockSpec(lin_W.shape, lambda i: (0, 0)),
                  pl.BlockSpec((1, 16), lambda i: (0, 0))],
        out_specs=pl.BlockSpec((G, 16), lambda i: (0, 0)),
        out_shape=jax.ShapeDtypeStruct((G, 16), jnp.float32),
        scratch_shapes=[pltpu.VMEM((G, 256), jnp.float32),
                        pltpu.VMEM((G, 128), jnp.float32)],
    )(agg, degp, W3, b3.reshape(1, -1), batch3, lin_W, lin_b.reshape(1, -1))


def kernel(x, edge_index, batch, W1, b1, W2, b2, W3, b3, lin_W, lin_b):
    src = edge_index[0]
    dst = edge_index[1]
    pad = EP - E
    src_p = jnp.concatenate([src, jnp.full((pad,), N, jnp.int32)])
    dst_p = jnp.concatenate([dst, jnp.full((pad,), N, jnp.int32)])
    src2 = (src_p.reshape(1, 16 * NCH, B)
            + jnp.array([0, NP], jnp.int32)[:, None, None]).reshape(-1, B)
    dst2 = dst_p.reshape(16 * NCH, B)
    srcd = src_p.reshape(32 * DCH, B)
    dstd = dst_p.reshape(32 * DCH, B)
    x_pad = jnp.concatenate([x, jnp.zeros((NP - N, 128), jnp.float32)])
    batch3 = jnp.concatenate(
        [batch, jnp.full((NP - N,), G, jnp.int32)]).reshape(NBLK, 1, BLK)

    degp = _sc_degree(dstd)
    table1 = _tc_prep(degp, x_pad)
    agg1 = _sc_agg_edges(table1, srcd, dstd)
    table2 = _tc_layer(agg1, degp, W1, b1, "sum")
    agg2 = _sc_agg_cols(table2.reshape(2 * NP, D), src2, dst2)
    table3 = _tc_layer(agg2.reshape(2, NP, D), degp, W2, b2, "concat")
    agg3 = _sc_agg_cols(table3.reshape(2 * NP, D), src2, dst2)
    return _tc_final(agg3.reshape(2, NP, D), degp, W3, b3,
                     batch3, lin_W, lin_b)
